# Initial kernel scaffold; baseline (speedup 1.0000x reference)
import jax
import jax.numpy as jnp
from jax.experimental import pallas as pl


def kernel(x, emb, W, b):
    raise NotImplementedError("write your pallas kernel here")


# speedup vs baseline: 10.0092x; 10.0092x over previous
import functools

import jax
import jax.numpy as jnp
from jax import lax
from jax.experimental import pallas as pl
from jax.experimental.pallas import tpu as pltpu
from jax.experimental.pallas import tpu_sc as plsc

NUM_EMB = 100000
DIM = 128
QMIN = -128.0
QMAX = 127.0
EPS = 1.1920929e-07

ROW_BLK = 1000
N_BLK = NUM_EMB // ROW_BLK

NC = 2
NS = 16
LANES = 16

_MESH = dict(core_axis_name="c", subcore_axis_name="s", num_cores=NC,
             num_subcores=NS)


def _rowminmax_body(emb_ref, omin_ref, omax_ref):
    blk = emb_ref[...]
    omin_ref[...] = jnp.min(blk, axis=1).reshape(1, 1, ROW_BLK)
    omax_ref[...] = jnp.max(blk, axis=1).reshape(1, 1, ROW_BLK)


def _rowminmax(emb):
    return pl.pallas_call(
        _rowminmax_body,
        grid=(N_BLK,),
        in_specs=[pl.BlockSpec((ROW_BLK, DIM), lambda i: (i, 0))],
        out_specs=[
            pl.BlockSpec((1, 1, ROW_BLK), lambda i: (i, 0, 0)),
            pl.BlockSpec((1, 1, ROW_BLK), lambda i: (i, 0, 0)),
        ],
        out_shape=[
            jax.ShapeDtypeStruct((N_BLK, 1, ROW_BLK), jnp.float32),
            jax.ShapeDtypeStruct((N_BLK, 1, ROW_BLK), jnp.float32),
        ],
    )(emb)


def _sc_minmax(xf, rowmin, rowmax):
    total = xf.shape[0]
    n_per_tile = total // NS
    chunk = 6400
    n_chunks = n_per_tile // chunk

    @functools.partial(
        pl.kernel,
        out_type=jax.ShapeDtypeStruct((NC, NS, LANES), jnp.float32),
        mesh=plsc.VectorSubcoreMesh(**_MESH),
        compiler_params=pltpu.CompilerParams(needs_layout_passes=False),
        scratch_types=[
            pltpu.VMEM((NUM_EMB,), jnp.float32),
            pltpu.VMEM((chunk,), jnp.int32),
            pltpu.VMEM((LANES,), jnp.float32),
        ],
    )
    def k(xf_hbm, rmin_hbm, rmax_hbm, out_hbm, tbl_v, idx_v, acc_v):
        cid = lax.axis_index("c")
        sid = lax.axis_index("s")
        is_min = cid == 0

        @pl.when(is_min)
        def _():
            pltpu.sync_copy(rmin_hbm, tbl_v)
            acc_v[...] = jnp.full((LANES,), jnp.inf, jnp.float32)

        @pl.when(jnp.logical_not(is_min))
        def _():
            pltpu.sync_copy(rmax_hbm, tbl_v)
            acc_v[...] = jnp.full((LANES,), -jnp.inf, jnp.float32)

        base0 = sid * n_per_tile

        def do_chunk(c, _):
            pltpu.sync_copy(xf_hbm.at[pl.ds(base0 + c * chunk, chunk)], idx_v)

            def step(j, _):
                iv = idx_v[pl.ds(j * LANES, LANES)]
                vals = plsc.load_gather(tbl_v, [iv])

                @pl.when(is_min)
                def _():
                    acc_v[...] = jnp.minimum(acc_v[...], vals)

                @pl.when(jnp.logical_not(is_min))
                def _():
                    acc_v[...] = jnp.maximum(acc_v[...], vals)

                return 0

            lax.fori_loop(0, chunk // LANES, step, 0)
            return 0

        lax.fori_loop(0, n_chunks, do_chunk, 0)
        pltpu.sync_copy(acc_v, out_hbm.at[cid, sid])

    return k(xf, rowmin, rowmax)


def _table_body(emb_ref, p_ref, w_ref, b_ref, out_ref):
    p = p_ref[...]
    min_val = jnp.minimum(jnp.min(p[0]), 0.0)
    max_val = jnp.maximum(jnp.max(p[1]), 0.0)
    scale = jnp.maximum((max_val - min_val) / (QMAX - QMIN), EPS)
    zp = jnp.clip(QMIN - jnp.round(min_val / scale), QMIN, QMAX)
    h = emb_ref[...]
    xq = jnp.round(h / scale) + zp
    y = (jnp.clip(xq, QMIN, QMAX) - zp) * scale
    t = jnp.sum(y * w_ref[0, :][None, :], axis=1) + b_ref[0, 0]
    out_ref[...] = t.reshape(1, 1, ROW_BLK)


def _table(emb, partials, W, b2):
    return pl.pallas_call(
        _table_body,
        grid=(N_BLK,),
        in_specs=[
            pl.BlockSpec((ROW_BLK, DIM), lambda i: (i, 0)),
            pl.BlockSpec((NC, NS, LANES), lambda i: (0, 0, 0)),
            pl.BlockSpec((1, DIM), lambda i: (0, 0)),
            pl.BlockSpec((1, 1), lambda i: (0, 0)),
        ],
        out_specs=pl.BlockSpec((1, 1, ROW_BLK), lambda i: (i, 0, 0)),
        out_shape=jax.ShapeDtypeStruct((N_BLK, 1, ROW_BLK), jnp.float32),
    )(emb, partials, W, b2)


def _sc_gather(xf, t):
    total = xf.shape[0]
    n_workers = NC * NS
    n_per = total // n_workers
    chunk = 6400
    n_chunks = n_per // chunk

    @functools.partial(
        pl.kernel,
        out_type=jax.ShapeDtypeStruct((total,), jnp.float32),
        mesh=plsc.VectorSubcoreMesh(**_MESH),
        compiler_params=pltpu.CompilerParams(needs_layout_passes=False),
        scratch_types=[
            pltpu.VMEM((NUM_EMB,), jnp.float32),
            pltpu.VMEM((chunk,), jnp.int32),
            pltpu.VMEM((chunk,), jnp.float32),
        ],
    )
    def k(xf_hbm, t_hbm, out_hbm, tbl_v, idx_v, val_v):
        cid = lax.axis_index("c")
        sid = lax.axis_index("s")
        wid = sid * NC + cid
        pltpu.sync_copy(t_hbm, tbl_v)
        base0 = wid * n_per

        def do_chunk(c, _):
            base = base0 + c * chunk
            pltpu.sync_copy(xf_hbm.at[pl.ds(base, chunk)], idx_v)

            def step(j, _):
                iv = idx_v[pl.ds(j * LANES, LANES)]
                val_v[pl.ds(j * LANES, LANES)] = plsc.load_gather(tbl_v, [iv])
                return 0

            lax.fori_loop(0, chunk // LANES, step, 0)
            pltpu.sync_copy(val_v, out_hbm.at[pl.ds(base, chunk)])
            return 0

        lax.fori_loop(0, n_chunks, do_chunk, 0)

    return k(xf, t)


def kernel(x, emb, W, b):
    bsz, seq = x.shape
    xf = x.reshape(-1)
    rowmin, rowmax = _rowminmax(emb)
    partials = _sc_minmax(xf, rowmin.reshape(-1), rowmax.reshape(-1))
    t = _table(emb, partials, W, b.reshape(1, 1))
    outf = _sc_gather(xf, t.reshape(-1))
    return outf.reshape(bsz, seq, 1)


# speedup vs baseline: 11.3011x; 1.1291x over previous
import functools

import jax
import jax.numpy as jnp
from jax import lax
from jax.experimental import pallas as pl
from jax.experimental.pallas import tpu as pltpu
from jax.experimental.pallas import tpu_sc as plsc

NUM_EMB = 100000
DIM = 128
QMIN = -128.0
QMAX = 127.0
EPS = 1.1920929e-07

ROW_BLK = 1000
N_BLK = NUM_EMB // ROW_BLK

NC = 2
NS = 16
LANES = 16

_MESH = dict(core_axis_name="c", subcore_axis_name="s", num_cores=NC,
             num_subcores=NS)


def _rowminmax_body(emb_ref, omin_ref, omax_ref):
    blk = emb_ref[...]
    omin_ref[...] = jnp.min(blk, axis=1).reshape(1, 1, ROW_BLK)
    omax_ref[...] = jnp.max(blk, axis=1).reshape(1, 1, ROW_BLK)


def _rowminmax(emb):
    return pl.pallas_call(
        _rowminmax_body,
        grid=(N_BLK,),
        in_specs=[pl.BlockSpec((ROW_BLK, DIM), lambda i: (i, 0))],
        out_specs=[
            pl.BlockSpec((1, 1, ROW_BLK), lambda i: (i, 0, 0)),
            pl.BlockSpec((1, 1, ROW_BLK), lambda i: (i, 0, 0)),
        ],
        out_shape=[
            jax.ShapeDtypeStruct((N_BLK, 1, ROW_BLK), jnp.float32),
            jax.ShapeDtypeStruct((N_BLK, 1, ROW_BLK), jnp.float32),
        ],
    )(emb)


def _sc_minmax(xf, rowmin, rowmax):
    total = xf.shape[0]
    n_per_tile = total // NS
    chunk = 25600
    n_chunks = n_per_tile // chunk
    unroll = 8

    @functools.partial(
        pl.kernel,
        out_type=jax.ShapeDtypeStruct((NC, NS, LANES), jnp.float32),
        mesh=plsc.VectorSubcoreMesh(**_MESH),
        compiler_params=pltpu.CompilerParams(needs_layout_passes=False),
        scratch_types=[
            pltpu.VMEM((NUM_EMB,), jnp.float32),
            pltpu.VMEM((chunk,), jnp.int32),
            pltpu.VMEM((LANES,), jnp.float32),
        ],
    )
    def k(xf_hbm, rmin_hbm, rmax_hbm, out_hbm, tbl_v, idx_v, acc_v):
        cid = lax.axis_index("c")
        sid = lax.axis_index("s")
        is_min = cid == 0
        base0 = sid * n_per_tile

        def run(op, init, tbl_hbm):
            pltpu.sync_copy(tbl_hbm, tbl_v)

            def do_chunk(c, acc):
                pltpu.sync_copy(xf_hbm.at[pl.ds(base0 + c * chunk, chunk)],
                                idx_v)

                def step(j, acc):
                    for u in range(unroll):
                        iv = idx_v[pl.ds((j * unroll + u) * LANES, LANES)]
                        acc = op(acc, plsc.load_gather(tbl_v, [iv]))
                    return acc

                return lax.fori_loop(0, chunk // (LANES * unroll), step, acc)

            acc = lax.fori_loop(0, n_chunks, do_chunk,
                                jnp.full((LANES,), init, jnp.float32))
            acc_v[...] = acc

        @pl.when(is_min)
        def _():
            run(jnp.minimum, jnp.inf, rmin_hbm)

        @pl.when(jnp.logical_not(is_min))
        def _():
            run(jnp.maximum, -jnp.inf, rmax_hbm)

        pltpu.sync_copy(acc_v, out_hbm.at[cid, sid])

    return k(xf, rowmin, rowmax)


def _table_body(emb_ref, p_ref, w_ref, b_ref, out_ref):
    p = p_ref[...]
    min_val = jnp.minimum(jnp.min(p[0]), 0.0)
    max_val = jnp.maximum(jnp.max(p[1]), 0.0)
    scale = jnp.maximum((max_val - min_val) / (QMAX - QMIN), EPS)
    zp = jnp.clip(QMIN - jnp.round(min_val / scale), QMIN, QMAX)
    h = emb_ref[...]
    xq = jnp.round(h / scale) + zp
    y = (jnp.clip(xq, QMIN, QMAX) - zp) * scale
    t = jnp.sum(y * w_ref[0, :][None, :], axis=1) + b_ref[0, 0]
    out_ref[...] = t.reshape(1, 1, ROW_BLK)


def _table(emb, partials, W, b2):
    return pl.pallas_call(
        _table_body,
        grid=(N_BLK,),
        in_specs=[
            pl.BlockSpec((ROW_BLK, DIM), lambda i: (i, 0)),
            pl.BlockSpec((NC, NS, LANES), lambda i: (0, 0, 0)),
            pl.BlockSpec((1, DIM), lambda i: (0, 0)),
            pl.BlockSpec((1, 1), lambda i: (0, 0)),
        ],
        out_specs=pl.BlockSpec((1, 1, ROW_BLK), lambda i: (i, 0, 0)),
        out_shape=jax.ShapeDtypeStruct((N_BLK, 1, ROW_BLK), jnp.float32),
    )(emb, partials, W, b2)


def _sc_gather(xf, t):
    total = xf.shape[0]
    n_workers = NC * NS
    n_per = total // n_workers
    chunk = 12800
    n_chunks = n_per // chunk
    unroll = 8

    @functools.partial(
        pl.kernel,
        out_type=jax.ShapeDtypeStruct((total,), jnp.float32),
        mesh=plsc.VectorSubcoreMesh(**_MESH),
        compiler_params=pltpu.CompilerParams(needs_layout_passes=False),
        scratch_types=[
            pltpu.VMEM((NUM_EMB,), jnp.float32),
            pltpu.VMEM((chunk,), jnp.int32),
            pltpu.VMEM((chunk,), jnp.float32),
        ],
    )
    def k(xf_hbm, t_hbm, out_hbm, tbl_v, idx_v, val_v):
        cid = lax.axis_index("c")
        sid = lax.axis_index("s")
        wid = sid * NC + cid
        pltpu.sync_copy(t_hbm, tbl_v)
        base0 = wid * n_per

        def do_chunk(c, _):
            base = base0 + c * chunk
            pltpu.sync_copy(xf_hbm.at[pl.ds(base, chunk)], idx_v)

            def step(j, _):
                for u in range(unroll):
                    off = (j * unroll + u) * LANES
                    iv = idx_v[pl.ds(off, LANES)]
                    val_v[pl.ds(off, LANES)] = plsc.load_gather(tbl_v, [iv])
                return 0

            lax.fori_loop(0, chunk // (LANES * unroll), step, 0)
            pltpu.sync_copy(val_v, out_hbm.at[pl.ds(base, chunk)])
            return 0

        lax.fori_loop(0, n_chunks, do_chunk, 0)

    return k(xf, t)


def kernel(x, emb, W, b):
    bsz, seq = x.shape
    xf = x.reshape(-1)
    rowmin, rowmax = _rowminmax(emb)
    partials = _sc_minmax(xf, rowmin.reshape(-1), rowmax.reshape(-1))
    t = _table(emb, partials, W, b.reshape(1, 1))
    outf = _sc_gather(xf, t.reshape(-1))
    return outf.reshape(bsz, seq, 1)
